# Initial kernel scaffold; baseline (speedup 1.0000x reference)
import jax
import jax.numpy as jnp
from jax.experimental import pallas as pl


def kernel(x, edge_index, edge_attr, batch, W1, as1, ad1, We1, ae1, b1, g1, bt1, W2, as2, ad2, We2, ae2, b2, gl1, bl1, Wl1, bb1, gl2, bl2, Wl2, bb2, gl3, bl3, Wl3, bb3, Wx, bx, Wy, by, Wr, br):
    raise NotImplementedError("write your pallas kernel here")


# speedup vs baseline: 30.4700x; 30.4700x over previous
import jax
import jax.numpy as jnp
from jax import lax
from jax.experimental import pallas as pl
from jax.experimental.pallas import tpu as pltpu
from jax.experimental.pallas import tpu_sc as plsc

N = 10000
E = 320000
D = 128
HID = 16
G = 128
ODX = 20
ODY = 20

NC, NS, L = 2, 16, 16
NW = NC * NS
EPW = 10240
E_PAD = NW * EPW
ROWS = E_PAD // 128
RPW = EPW // 128
CHUNK_ROWS = 8
C = CHUNK_ROWS * 128
NCHUNK = RPW // CHUNK_ROWS
N_ACC = 10240
RPT = N_ACC // NS
NEG = -1e30
ACC_W = 32


def _edge_body(h_hbm, an_hbm, ad_hbm, src_hbm, dst_hbm, ea_hbm, zz_hbm,
               out_hbm, an_v, ad_v, hbuf, sbuf, srcv, dstv, eav, acc, sem):
    cid = lax.axis_index("c")
    sid = lax.axis_index("s")
    wid = sid * NC + cid

    pltpu.sync_copy(an_hbm, an_v)
    pltpu.sync_copy(ad_hbm, ad_v)
    pltpu.sync_copy(zz_hbm.at[pl.ds(sid * RPT, RPT)],
                    acc.at[pl.ds(sid * RPT, RPT)])

    def _z(r, carry):
        sbuf[r, pl.ds(HID, 16)] = jnp.zeros((16,), jnp.float32)
        return carry
    lax.fori_loop(0, C, _z, 0)

    plsc.subcore_barrier()

    row00 = wid * RPW

    def chunk(chi, carry):
        row0 = row00 + chi * CHUNK_ROWS
        pltpu.sync_copy(src_hbm.at[pl.ds(row0, CHUNK_ROWS)], srcv)
        pltpu.sync_copy(dst_hbm.at[pl.ds(row0, CHUNK_ROWS)], dstv)
        pltpu.sync_copy(ea_hbm.at[pl.ds(row0, CHUNK_ROWS)], eav)
        descs = [pltpu.async_copy(h_hbm.at[srcv.at[i]],
                                  hbuf.at[pl.ds(i * 128, 128)], sem)
                 for i in range(CHUNK_ROWS)]
        for d in descs:
            d.wait()
        for i in range(CHUNK_ROWS):
            for k in range(8):
                si = srcv[i, pl.ds(k * 16, 16)]
                di = dstv[i, pl.ds(k * 16, 16)]
                ev = eav[i, pl.ds(k * 16, 16)]
                al = plsc.load_gather(an_v, [si]) \
                    + plsc.load_gather(ad_v, [di]) + ev
                al = jnp.where(al > 0, al, 0.2 * al)
                p = jnp.exp(al)
                rows = lax.iota(jnp.int32, 16) + (i * 128 + k * 16)
                for c in range(HID):
                    col = jnp.full((16,), c, jnp.int32)
                    hv = plsc.load_gather(hbuf, [rows, col])
                    plsc.store_scatter(sbuf, [rows, col], hv * p)
                plsc.store_scatter(sbuf, [rows, jnp.full((16,), HID, jnp.int32)], p)
        for i in range(CHUNK_ROWS):
            pltpu.sync_copy(sbuf.at[pl.ds(i * 128, 128)],
                            acc.at[dstv.at[i]], add=True)
        return carry

    lax.fori_loop(0, NCHUNK, chunk, 0)
    plsc.subcore_barrier()
    pltpu.sync_copy(acc.at[pl.ds(sid * RPT, RPT)],
                    out_hbm.at[cid, pl.ds(sid * RPT, RPT)])


_edge_kernel = pl.kernel(
    _edge_body,
    out_type=jax.ShapeDtypeStruct((NC, N_ACC, ACC_W), jnp.float32),
    mesh=plsc.VectorSubcoreMesh(core_axis_name="c", subcore_axis_name="s",
                                num_cores=NC, num_subcores=NS),
    compiler_params=pltpu.CompilerParams(needs_layout_passes=False,
                                         use_tc_tiling_on_sc=False),
    scratch_types=[
        pltpu.VMEM((N,), jnp.float32),
        pltpu.VMEM((N,), jnp.float32),
        pltpu.VMEM((C, HID), jnp.float32),
        pltpu.VMEM((C, ACC_W), jnp.float32),
        pltpu.VMEM((CHUNK_ROWS, 128), jnp.int32),
        pltpu.VMEM((CHUNK_ROWS, 128), jnp.int32),
        pltpu.VMEM((CHUNK_ROWS, 128), jnp.float32),
        pltpu.VMEM_SHARED((N_ACC, ACC_W), jnp.float32),
        pltpu.SemaphoreType.DMA,
    ],
)


def _prep_body(x_ref, W1_ref, as1_ref, ad1_ref, We1_ref, ae1_ref, We2_ref,
               ae2_ref, eaT_ref, h_ref, an_ref, ad_ref, ea1_ref, ea2_ref):
    h = jnp.dot(x_ref[...], W1_ref[...], preferred_element_type=jnp.float32)
    h_ref[...] = h
    an_ref[...] = jnp.sum(h * as1_ref[...][None, :], axis=1)
    ad_ref[...] = jnp.sum(h * ad1_ref[...][None, :], axis=1)
    ea = eaT_ref[...]
    r = lax.broadcasted_iota(jnp.int32, (ROWS, 128), 0)
    cc = lax.broadcasted_iota(jnp.int32, (ROWS, 128), 1)
    valid = (r * 128 + cc) < E
    c1 = jnp.sum(We1_ref[...] * ae1_ref[...][None, :], axis=1)
    c2 = jnp.sum(We2_ref[...] * ae2_ref[...][None, :], axis=1)
    ea1 = (ea * c1[:, None, None]).sum(axis=0)
    ea2 = (ea * c2[:, None, None]).sum(axis=0)
    ea1_ref[...] = jnp.where(valid, ea1, NEG)
    ea2_ref[...] = jnp.where(valid, ea2, NEG)


_prep_call = pl.pallas_call(
    _prep_body,
    out_shape=[
        jax.ShapeDtypeStruct((N, HID), jnp.float32),
        jax.ShapeDtypeStruct((N,), jnp.float32),
        jax.ShapeDtypeStruct((N,), jnp.float32),
        jax.ShapeDtypeStruct((ROWS, 128), jnp.float32),
        jax.ShapeDtypeStruct((ROWS, 128), jnp.float32),
    ],
)


def _mid_body(part_ref, b1_ref, g1_ref, bt1_ref, W2_ref, as2_ref, ad2_ref,
              h2_ref, an2_ref, adn2_ref):
    num = part_ref[0, 0:N, 0:HID] + part_ref[1, 0:N, 0:HID]
    s = part_ref[0, 0:N, HID:HID + 1] + part_ref[1, 0:N, HID:HID + 1]
    h1 = jax.nn.relu(num / jnp.maximum(s, 1e-16) + b1_ref[...][None, :])
    m = jnp.mean(h1, axis=0)
    var = jnp.mean((h1 - m[None, :]) ** 2, axis=0)
    hn = (h1 - m[None, :]) / jnp.sqrt(var + 1e-5)[None, :] \
        * g1_ref[...][None, :] + bt1_ref[...][None, :]
    h2 = jnp.dot(hn, W2_ref[...], preferred_element_type=jnp.float32)
    h2_ref[...] = h2
    an2_ref[...] = jnp.sum(h2 * as2_ref[...][None, :], axis=1)
    adn2_ref[...] = jnp.sum(h2 * ad2_ref[...][None, :], axis=1)


_mid_call = pl.pallas_call(
    _mid_body,
    out_shape=[
        jax.ShapeDtypeStruct((N, HID), jnp.float32),
        jax.ShapeDtypeStruct((N,), jnp.float32),
        jax.ShapeDtypeStruct((N,), jnp.float32),
    ],
)


def _bn_rows(v, g, b):
    m = jnp.mean(v, axis=0)
    var = jnp.mean((v - m[None, :]) ** 2, axis=0)
    return (v - m[None, :]) / jnp.sqrt(var + 1e-5)[None, :] * g[None, :] \
        + b[None, :]


def _softmax_rows(z):
    e = jnp.exp(z - jnp.max(z, axis=1, keepdims=True))
    return e / jnp.sum(e, axis=1, keepdims=True)


def _final_body(part_ref, b2_ref, batch_ref, gl1_ref, bl1_ref, Wl1_ref,
                bb1_ref, gl2_ref, bl2_ref, Wl2_ref, bb2_ref, gl3_ref,
                bl3_ref, Wl3_ref, bb3_ref, Wx_ref, bx_ref, Wy_ref, by_ref,
                Wr_ref, br_ref, xx_ref, yy_ref, rot_ref):
    num = part_ref[0, 0:N, 0:HID] + part_ref[1, 0:N, 0:HID]
    s = part_ref[0, 0:N, HID:HID + 1] + part_ref[1, 0:N, HID:HID + 1]
    hf = jax.nn.relu(num / jnp.maximum(s, 1e-16) + b2_ref[...][None, :])
    gm = lax.broadcasted_iota(jnp.int32, (N, G), 1)
    M = jnp.where(gm == batch_ref[...][:, None], 1.0, 0.0)
    enc_sum = lax.dot_general(M, hf, (((0,), (0,)), ((), ())),
                              preferred_element_type=jnp.float32)
    cnt = jnp.sum(M, axis=0)
    enc = enc_sum / jnp.maximum(cnt, 1.0)[:, None]
    t = _bn_rows(enc, gl1_ref[...], bl1_ref[...])
    t = jax.nn.relu(jnp.dot(t, Wl1_ref[...]) + bb1_ref[...][None, :])
    t = _bn_rows(jnp.concatenate([t, enc], axis=1), gl2_ref[...], bl2_ref[...])
    t = jax.nn.relu(jnp.dot(t, Wl2_ref[...]) + bb2_ref[...][None, :])
    t = _bn_rows(jnp.concatenate([t, enc], axis=1), gl3_ref[...], bl3_ref[...])
    t = jax.nn.relu(jnp.dot(t, Wl3_ref[...]) + bb3_ref[...][None, :])
    xx_ref[...] = _softmax_rows(jnp.dot(t, Wx_ref[...]) + bx_ref[...][None, :])
    yy_ref[...] = _softmax_rows(jnp.dot(t, Wy_ref[...]) + by_ref[...][None, :])
    rot_ref[...] = _softmax_rows(jnp.dot(t, Wr_ref[...]) + br_ref[...][None, :])


_final_call = pl.pallas_call(
    _final_body,
    out_shape=[
        jax.ShapeDtypeStruct((G, ODX), jnp.float32),
        jax.ShapeDtypeStruct((G, ODY), jnp.float32),
        jax.ShapeDtypeStruct((G, 4), jnp.float32),
    ],
)


def kernel(x, edge_index, edge_attr, batch, W1, as1, ad1, We1, ae1, b1, g1,
           bt1, W2, as2, ad2, We2, ae2, b2, gl1, bl1, Wl1, bb1, gl2, bl2,
           Wl2, bb2, gl3, bl3, Wl3, bb3, Wx, bx, Wy, by, Wr, br):
    pad = E_PAD - E
    src = jnp.concatenate([edge_index[0], jnp.zeros((pad,), jnp.int32)])
    dst = jnp.concatenate([edge_index[1], jnp.zeros((pad,), jnp.int32)])
    srcp = src.reshape(ROWS, 128)
    dstp = dst.reshape(ROWS, 128)
    eaT = jnp.concatenate(
        [edge_attr, jnp.zeros((pad, 2), jnp.float32)], axis=0
    ).T.reshape(2, ROWS, 128)
    zz = jnp.zeros((N_ACC, ACC_W), jnp.float32)

    h1, an1, adn1, ea1p, ea2p = _prep_call(x, W1, as1, ad1, We1, ae1, We2,
                                           ae2, eaT)
    part1 = _edge_kernel(h1, an1, adn1, srcp, dstp, ea1p, zz)
    h2, an2, adn2 = _mid_call(part1, b1, g1, bt1, W2, as2, ad2)
    part2 = _edge_kernel(h2, an2, adn2, srcp, dstp, ea2p, zz)
    xx, yy, rot = _final_call(part2, b2, batch, gl1, bl1, Wl1, bb1, gl2,
                              bl2, Wl2, bb2, gl3, bl3, Wl3, bb3, Wx, bx,
                              Wy, by, Wr, br)
    return (xx, yy, rot)


# speedup vs baseline: 36.6756x; 1.2037x over previous
import jax
import jax.numpy as jnp
from jax import lax
from jax.experimental import pallas as pl
from jax.experimental.pallas import tpu as pltpu
from jax.experimental.pallas import tpu_sc as plsc

N = 10000
E = 320000
D = 128
HID = 16
G = 128
ODX = 20
ODY = 20

NC, NS, L = 2, 16, 16
NW = NC * NS
EPW = 10240
E_PAD = NW * EPW
ROWS = E_PAD // 128
RPW = EPW // 128
CHUNK_ROWS = 8
C = CHUNK_ROWS * 128
NCHUNK = RPW // CHUNK_ROWS
N_ACC = 10240
RPT = N_ACC // NS
NEG = -1e30
ACC_W = 32


def _edge_body(h_hbm, an_hbm, ad_hbm, src_hbm, dst_hbm, ea_hbm, zz_hbm,
               out_hbm, an_v, ad_v, hbuf, sbuf, srcv, dstv, eav, acc, gsem,
               ssem):
    cid = lax.axis_index("c")
    sid = lax.axis_index("s")
    wid = sid * NC + cid

    pltpu.sync_copy(an_hbm, an_v)
    pltpu.sync_copy(ad_hbm, ad_v)
    pltpu.sync_copy(zz_hbm.at[pl.ds(sid * RPT, RPT)],
                    acc.at[pl.ds(sid * RPT, RPT)])

    def _z(r, carry):
        sbuf[r, pl.ds(HID, 16)] = jnp.zeros((16,), jnp.float32)
        return carry
    lax.fori_loop(0, C, _z, 0)

    plsc.subcore_barrier()

    row00 = wid * RPW

    def load_idx_and_fire(chi, b):
        row0 = row00 + chi * CHUNK_ROWS
        pltpu.sync_copy(src_hbm.at[pl.ds(row0, CHUNK_ROWS)], srcv.at[b])
        pltpu.sync_copy(dst_hbm.at[pl.ds(row0, CHUNK_ROWS)], dstv.at[b])
        pltpu.sync_copy(ea_hbm.at[pl.ds(row0, CHUNK_ROWS)], eav.at[b])
        for i in range(CHUNK_ROWS):
            pltpu.make_async_copy(h_hbm.at[srcv.at[b, i]],
                                  hbuf.at[b, pl.ds(i * 128, 128)],
                                  gsem).start()

    load_idx_and_fire(0, 0)

    def step(chi, carry):
        b = lax.rem(chi, 2)
        nb = 1 - b

        @pl.when(chi < NCHUNK)
        def _wait_gathers():
            for i in range(CHUNK_ROWS):
                pltpu.make_async_copy(h_hbm.at[srcv.at[b, i]],
                                      hbuf.at[b, pl.ds(i * 128, 128)],
                                      gsem).wait()

        @pl.when(jnp.logical_and(chi >= 1, chi <= NCHUNK))
        def _wait_scatters():
            for i in range(CHUNK_ROWS):
                pltpu.make_async_copy(sbuf.at[pl.ds(i * 128, 128)],
                                      acc.at[dstv.at[nb, i]], ssem).wait()

        @pl.when(chi + 1 < NCHUNK)
        def _prefetch():
            load_idx_and_fire(chi + 1, nb)

        @pl.when(chi < NCHUNK)
        def _compute():
            for i in range(CHUNK_ROWS):
                for k in range(8):
                    si = srcv[b, i, pl.ds(k * 16, 16)]
                    di = dstv[b, i, pl.ds(k * 16, 16)]
                    ev = eav[b, i, pl.ds(k * 16, 16)]
                    al = plsc.load_gather(an_v, [si]) \
                        + plsc.load_gather(ad_v, [di]) + ev
                    al = jnp.where(al > 0, al, 0.2 * al)
                    p = jnp.exp(al)
                    rows = lax.iota(jnp.int32, 16) + (i * 128 + k * 16)
                    for c in range(HID):
                        col = jnp.full((16,), c, jnp.int32)
                        hv = plsc.load_gather(hbuf.at[b], [rows, col])
                        plsc.store_scatter(sbuf, [rows, col], hv * p)
                    plsc.store_scatter(sbuf,
                                       [rows, jnp.full((16,), HID, jnp.int32)],
                                       p)
            for i in range(CHUNK_ROWS):
                pltpu.make_async_copy(sbuf.at[pl.ds(i * 128, 128)],
                                      acc.at[dstv.at[b, i]],
                                      ssem).start(add=True)
        return carry

    lax.fori_loop(0, NCHUNK + 1, step, 0)
    plsc.subcore_barrier()
    pltpu.sync_copy(acc.at[pl.ds(sid * RPT, RPT)],
                    out_hbm.at[cid, pl.ds(sid * RPT, RPT)])


_edge_kernel = pl.kernel(
    _edge_body,
    out_type=jax.ShapeDtypeStruct((NC, N_ACC, ACC_W), jnp.float32),
    mesh=plsc.VectorSubcoreMesh(core_axis_name="c", subcore_axis_name="s",
                                num_cores=NC, num_subcores=NS),
    compiler_params=pltpu.CompilerParams(needs_layout_passes=False,
                                         use_tc_tiling_on_sc=False),
    scratch_types=[
        pltpu.VMEM((N,), jnp.float32),
        pltpu.VMEM((N,), jnp.float32),
        pltpu.VMEM((2, C, HID), jnp.float32),
        pltpu.VMEM((C, ACC_W), jnp.float32),
        pltpu.VMEM((2, CHUNK_ROWS, 128), jnp.int32),
        pltpu.VMEM((2, CHUNK_ROWS, 128), jnp.int32),
        pltpu.VMEM((2, CHUNK_ROWS, 128), jnp.float32),
        pltpu.VMEM_SHARED((N_ACC, ACC_W), jnp.float32),
        pltpu.SemaphoreType.DMA,
        pltpu.SemaphoreType.DMA,
    ],
)


def _prep_body(x_ref, W1_ref, as1_ref, ad1_ref, We1_ref, ae1_ref, We2_ref,
               ae2_ref, eaT_ref, h_ref, an_ref, ad_ref, ea1_ref, ea2_ref):
    h = jnp.dot(x_ref[...], W1_ref[...], preferred_element_type=jnp.float32)
    h_ref[...] = h
    an_ref[...] = jnp.sum(h * as1_ref[...][None, :], axis=1)
    ad_ref[...] = jnp.sum(h * ad1_ref[...][None, :], axis=1)
    ea = eaT_ref[...]
    r = lax.broadcasted_iota(jnp.int32, (ROWS, 128), 0)
    cc = lax.broadcasted_iota(jnp.int32, (ROWS, 128), 1)
    valid = (r * 128 + cc) < E
    c1 = jnp.sum(We1_ref[...] * ae1_ref[...][None, :], axis=1)
    c2 = jnp.sum(We2_ref[...] * ae2_ref[...][None, :], axis=1)
    ea1 = (ea * c1[:, None, None]).sum(axis=0)
    ea2 = (ea * c2[:, None, None]).sum(axis=0)
    ea1_ref[...] = jnp.where(valid, ea1, NEG)
    ea2_ref[...] = jnp.where(valid, ea2, NEG)


_prep_call = pl.pallas_call(
    _prep_body,
    out_shape=[
        jax.ShapeDtypeStruct((N, HID), jnp.float32),
        jax.ShapeDtypeStruct((N,), jnp.float32),
        jax.ShapeDtypeStruct((N,), jnp.float32),
        jax.ShapeDtypeStruct((ROWS, 128), jnp.float32),
        jax.ShapeDtypeStruct((ROWS, 128), jnp.float32),
    ],
)


def _mid_body(part_ref, b1_ref, g1_ref, bt1_ref, W2_ref, as2_ref, ad2_ref,
              h2_ref, an2_ref, adn2_ref):
    num = part_ref[0, 0:N, 0:HID] + part_ref[1, 0:N, 0:HID]
    s = part_ref[0, 0:N, HID:HID + 1] + part_ref[1, 0:N, HID:HID + 1]
    h1 = jax.nn.relu(num / jnp.maximum(s, 1e-16) + b1_ref[...][None, :])
    m = jnp.mean(h1, axis=0)
    var = jnp.mean((h1 - m[None, :]) ** 2, axis=0)
    hn = (h1 - m[None, :]) / jnp.sqrt(var + 1e-5)[None, :] \
        * g1_ref[...][None, :] + bt1_ref[...][None, :]
    h2 = jnp.dot(hn, W2_ref[...], preferred_element_type=jnp.float32)
    h2_ref[...] = h2
    an2_ref[...] = jnp.sum(h2 * as2_ref[...][None, :], axis=1)
    adn2_ref[...] = jnp.sum(h2 * ad2_ref[...][None, :], axis=1)


_mid_call = pl.pallas_call(
    _mid_body,
    out_shape=[
        jax.ShapeDtypeStruct((N, HID), jnp.float32),
        jax.ShapeDtypeStruct((N,), jnp.float32),
        jax.ShapeDtypeStruct((N,), jnp.float32),
    ],
)


def _bn_rows(v, g, b):
    m = jnp.mean(v, axis=0)
    var = jnp.mean((v - m[None, :]) ** 2, axis=0)
    return (v - m[None, :]) / jnp.sqrt(var + 1e-5)[None, :] * g[None, :] \
        + b[None, :]


def _softmax_rows(z):
    e = jnp.exp(z - jnp.max(z, axis=1, keepdims=True))
    return e / jnp.sum(e, axis=1, keepdims=True)


def _final_body(part_ref, b2_ref, batch_ref, gl1_ref, bl1_ref, Wl1_ref,
                bb1_ref, gl2_ref, bl2_ref, Wl2_ref, bb2_ref, gl3_ref,
                bl3_ref, Wl3_ref, bb3_ref, Wx_ref, bx_ref, Wy_ref, by_ref,
                Wr_ref, br_ref, xx_ref, yy_ref, rot_ref):
    num = part_ref[0, 0:N, 0:HID] + part_ref[1, 0:N, 0:HID]
    s = part_ref[0, 0:N, HID:HID + 1] + part_ref[1, 0:N, HID:HID + 1]
    hf = jax.nn.relu(num / jnp.maximum(s, 1e-16) + b2_ref[...][None, :])
    gm = lax.broadcasted_iota(jnp.int32, (N, G), 1)
    M = jnp.where(gm == batch_ref[...][:, None], 1.0, 0.0)
    enc_sum = lax.dot_general(M, hf, (((0,), (0,)), ((), ())),
                              preferred_element_type=jnp.float32)
    cnt = jnp.sum(M, axis=0)
    enc = enc_sum / jnp.maximum(cnt, 1.0)[:, None]
    t = _bn_rows(enc, gl1_ref[...], bl1_ref[...])
    t = jax.nn.relu(jnp.dot(t, Wl1_ref[...]) + bb1_ref[...][None, :])
    t = _bn_rows(jnp.concatenate([t, enc], axis=1), gl2_ref[...], bl2_ref[...])
    t = jax.nn.relu(jnp.dot(t, Wl2_ref[...]) + bb2_ref[...][None, :])
    t = _bn_rows(jnp.concatenate([t, enc], axis=1), gl3_ref[...], bl3_ref[...])
    t = jax.nn.relu(jnp.dot(t, Wl3_ref[...]) + bb3_ref[...][None, :])
    xx_ref[...] = _softmax_rows(jnp.dot(t, Wx_ref[...]) + bx_ref[...][None, :])
    yy_ref[...] = _softmax_rows(jnp.dot(t, Wy_ref[...]) + by_ref[...][None, :])
    rot_ref[...] = _softmax_rows(jnp.dot(t, Wr_ref[...]) + br_ref[...][None, :])


_final_call = pl.pallas_call(
    _final_body,
    out_shape=[
        jax.ShapeDtypeStruct((G, ODX), jnp.float32),
        jax.ShapeDtypeStruct((G, ODY), jnp.float32),
        jax.ShapeDtypeStruct((G, 4), jnp.float32),
    ],
)


def kernel(x, edge_index, edge_attr, batch, W1, as1, ad1, We1, ae1, b1, g1,
           bt1, W2, as2, ad2, We2, ae2, b2, gl1, bl1, Wl1, bb1, gl2, bl2,
           Wl2, bb2, gl3, bl3, Wl3, bb3, Wx, bx, Wy, by, Wr, br):
    pad = E_PAD - E
    src = jnp.concatenate([edge_index[0], jnp.zeros((pad,), jnp.int32)])
    dst = jnp.concatenate([edge_index[1], jnp.zeros((pad,), jnp.int32)])
    srcp = src.reshape(ROWS, 128)
    dstp = dst.reshape(ROWS, 128)
    eaT = jnp.concatenate(
        [edge_attr, jnp.zeros((pad, 2), jnp.float32)], axis=0
    ).T.reshape(2, ROWS, 128)
    zz = jnp.zeros((N_ACC, ACC_W), jnp.float32)

    h1, an1, adn1, ea1p, ea2p = _prep_call(x, W1, as1, ad1, We1, ae1, We2,
                                           ae2, eaT)
    part1 = _edge_kernel(h1, an1, adn1, srcp, dstp, ea1p, zz)
    h2, an2, adn2 = _mid_call(part1, b1, g1, bt1, W2, as2, ad2)
    part2 = _edge_kernel(h2, an2, adn2, srcp, dstp, ea2p, zz)
    xx, yy, rot = _final_call(part2, b2, batch, gl1, bl1, Wl1, bb1, gl2,
                              bl2, Wl2, bb2, gl3, bl3, Wl3, bb3, Wx, bx,
                              Wy, by, Wr, br)
    return (xx, yy, rot)


# speedup vs baseline: 48.5444x; 1.3236x over previous
import jax
import jax.numpy as jnp
from jax import lax
from jax.experimental import pallas as pl
from jax.experimental.pallas import tpu as pltpu
from jax.experimental.pallas import tpu_sc as plsc

N = 10000
E = 320000
D = 128
HID = 16
G = 128
ODX = 20
ODY = 20

NC, NS, L = 2, 16, 16
NW = NC * NS
EPW = 10240
E_PAD = NW * EPW
ROWS = E_PAD // 128
RPW = EPW // 128
CHUNK_ROWS = 8
C = CHUNK_ROWS * 128
NCHUNK = RPW // CHUNK_ROWS
N_ACC = 10240
RPT = N_ACC // NS
NEG = -1e30
ACC_W = 32

_BCAST_DN = lax.GatherDimensionNumbers(
    offset_dims=(), collapsed_slice_dims=(0,), start_index_map=(0,))


def _lane_bcast(v, j):
    idx = jnp.full((16, 1), j, jnp.int32)
    return lax.gather(v, idx, _BCAST_DN, slice_sizes=(1,),
                      mode=lax.GatherScatterMode.PROMISE_IN_BOUNDS)


def _edge_body(h_hbm, an_hbm, ad_hbm, src_hbm, dst_hbm, ea_hbm, zz_hbm,
               out_hbm, an_v, ad_v, hbuf, sbuf, srcv, dstv, eav, acc, gsem,
               ssem):
    cid = lax.axis_index("c")
    sid = lax.axis_index("s")
    wid = sid * NC + cid

    pltpu.sync_copy(an_hbm, an_v)
    pltpu.sync_copy(ad_hbm, ad_v)
    pltpu.sync_copy(zz_hbm.at[pl.ds(sid * RPT, RPT)],
                    acc.at[pl.ds(sid * RPT, RPT)])

    def _z(r, carry):
        sbuf[r, pl.ds(HID, 16)] = jnp.zeros((16,), jnp.float32)
        return carry
    lax.fori_loop(0, C, _z, 0)

    plsc.subcore_barrier()

    row00 = wid * RPW

    def load_idx_and_fire(chi, b):
        row0 = row00 + chi * CHUNK_ROWS
        pltpu.sync_copy(src_hbm.at[pl.ds(row0, CHUNK_ROWS)], srcv.at[b])
        pltpu.sync_copy(dst_hbm.at[pl.ds(row0, CHUNK_ROWS)], dstv.at[b])
        pltpu.sync_copy(ea_hbm.at[pl.ds(row0, CHUNK_ROWS)], eav.at[b])
        for i in range(CHUNK_ROWS):
            pltpu.make_async_copy(h_hbm.at[srcv.at[b, i]],
                                  hbuf.at[b, pl.ds(i * 128, 128)],
                                  gsem).start()

    load_idx_and_fire(0, 0)

    def step(chi, carry):
        b = lax.rem(chi, 2)
        nb = 1 - b

        @pl.when(chi < NCHUNK)
        def _wait_gathers():
            for i in range(CHUNK_ROWS):
                pltpu.make_async_copy(h_hbm.at[srcv.at[b, i]],
                                      hbuf.at[b, pl.ds(i * 128, 128)],
                                      gsem).wait()

        @pl.when(jnp.logical_and(chi >= 1, chi <= NCHUNK))
        def _wait_scatters():
            for i in range(CHUNK_ROWS):
                pltpu.make_async_copy(sbuf.at[pl.ds(i * 128, 128)],
                                      acc.at[dstv.at[nb, i]], ssem).wait()

        @pl.when(chi + 1 < NCHUNK)
        def _prefetch():
            load_idx_and_fire(chi + 1, nb)

        @pl.when(chi < NCHUNK)
        def _compute():
            for i in range(CHUNK_ROWS):
                for k in range(8):
                    si = srcv[b, i, pl.ds(k * 16, 16)]
                    di = dstv[b, i, pl.ds(k * 16, 16)]
                    ev = eav[b, i, pl.ds(k * 16, 16)]
                    al = plsc.load_gather(an_v, [si]) \
                        + plsc.load_gather(ad_v, [di]) + ev
                    al = jnp.where(al > 0, al, 0.2 * al)
                    p = jnp.exp(al)
                    off = i * 128 + k * 16
                    for j in range(16):
                        pj = _lane_bcast(p, j)
                        hrow = hbuf[b, off + j, pl.ds(0, HID)]
                        sbuf[off + j, pl.ds(0, HID)] = hrow * pj
                    rows = lax.iota(jnp.int32, 16) + off
                    plsc.store_scatter(sbuf,
                                       [rows, jnp.full((16,), HID, jnp.int32)],
                                       p)
            for i in range(CHUNK_ROWS):
                pltpu.make_async_copy(sbuf.at[pl.ds(i * 128, 128)],
                                      acc.at[dstv.at[b, i]],
                                      ssem).start(add=True)
        return carry

    lax.fori_loop(0, NCHUNK + 1, step, 0)
    plsc.subcore_barrier()
    pltpu.sync_copy(acc.at[pl.ds(sid * RPT, RPT)],
                    out_hbm.at[cid, pl.ds(sid * RPT, RPT)])


_edge_kernel = pl.kernel(
    _edge_body,
    out_type=jax.ShapeDtypeStruct((NC, N_ACC, ACC_W), jnp.float32),
    mesh=plsc.VectorSubcoreMesh(core_axis_name="c", subcore_axis_name="s",
                                num_cores=NC, num_subcores=NS),
    compiler_params=pltpu.CompilerParams(needs_layout_passes=False,
                                         use_tc_tiling_on_sc=False),
    scratch_types=[
        pltpu.VMEM((N,), jnp.float32),
        pltpu.VMEM((N,), jnp.float32),
        pltpu.VMEM((2, C, HID), jnp.float32),
        pltpu.VMEM((C, ACC_W), jnp.float32),
        pltpu.VMEM((2, CHUNK_ROWS, 128), jnp.int32),
        pltpu.VMEM((2, CHUNK_ROWS, 128), jnp.int32),
        pltpu.VMEM((2, CHUNK_ROWS, 128), jnp.float32),
        pltpu.VMEM_SHARED((N_ACC, ACC_W), jnp.float32),
        pltpu.SemaphoreType.DMA,
        pltpu.SemaphoreType.DMA,
    ],
)


def _prep_body(x_ref, W1_ref, as1_ref, ad1_ref, We1_ref, ae1_ref, We2_ref,
               ae2_ref, eaT_ref, h_ref, an_ref, ad_ref, ea1_ref, ea2_ref):
    h = jnp.dot(x_ref[...], W1_ref[...], preferred_element_type=jnp.float32)
    h_ref[...] = h
    an_ref[...] = jnp.sum(h * as1_ref[...][None, :], axis=1)
    ad_ref[...] = jnp.sum(h * ad1_ref[...][None, :], axis=1)
    ea = eaT_ref[...]
    r = lax.broadcasted_iota(jnp.int32, (ROWS, 128), 0)
    cc = lax.broadcasted_iota(jnp.int32, (ROWS, 128), 1)
    valid = (r * 128 + cc) < E
    c1 = jnp.sum(We1_ref[...] * ae1_ref[...][None, :], axis=1)
    c2 = jnp.sum(We2_ref[...] * ae2_ref[...][None, :], axis=1)
    ea1 = (ea * c1[:, None, None]).sum(axis=0)
    ea2 = (ea * c2[:, None, None]).sum(axis=0)
    ea1_ref[...] = jnp.where(valid, ea1, NEG)
    ea2_ref[...] = jnp.where(valid, ea2, NEG)


_prep_call = pl.pallas_call(
    _prep_body,
    out_shape=[
        jax.ShapeDtypeStruct((N, HID), jnp.float32),
        jax.ShapeDtypeStruct((N,), jnp.float32),
        jax.ShapeDtypeStruct((N,), jnp.float32),
        jax.ShapeDtypeStruct((ROWS, 128), jnp.float32),
        jax.ShapeDtypeStruct((ROWS, 128), jnp.float32),
    ],
)


def _mid_body(part_ref, b1_ref, g1_ref, bt1_ref, W2_ref, as2_ref, ad2_ref,
              h2_ref, an2_ref, adn2_ref):
    num = part_ref[0, 0:N, 0:HID] + part_ref[1, 0:N, 0:HID]
    s = part_ref[0, 0:N, HID:HID + 1] + part_ref[1, 0:N, HID:HID + 1]
    h1 = jax.nn.relu(num / jnp.maximum(s, 1e-16) + b1_ref[...][None, :])
    m = jnp.mean(h1, axis=0)
    var = jnp.mean((h1 - m[None, :]) ** 2, axis=0)
    hn = (h1 - m[None, :]) / jnp.sqrt(var + 1e-5)[None, :] \
        * g1_ref[...][None, :] + bt1_ref[...][None, :]
    h2 = jnp.dot(hn, W2_ref[...], preferred_element_type=jnp.float32)
    h2_ref[...] = h2
    an2_ref[...] = jnp.sum(h2 * as2_ref[...][None, :], axis=1)
    adn2_ref[...] = jnp.sum(h2 * ad2_ref[...][None, :], axis=1)


_mid_call = pl.pallas_call(
    _mid_body,
    out_shape=[
        jax.ShapeDtypeStruct((N, HID), jnp.float32),
        jax.ShapeDtypeStruct((N,), jnp.float32),
        jax.ShapeDtypeStruct((N,), jnp.float32),
    ],
)


def _bn_rows(v, g, b):
    m = jnp.mean(v, axis=0)
    var = jnp.mean((v - m[None, :]) ** 2, axis=0)
    return (v - m[None, :]) / jnp.sqrt(var + 1e-5)[None, :] * g[None, :] \
        + b[None, :]


def _softmax_rows(z):
    e = jnp.exp(z - jnp.max(z, axis=1, keepdims=True))
    return e / jnp.sum(e, axis=1, keepdims=True)


def _final_body(part_ref, b2_ref, batch_ref, gl1_ref, bl1_ref, Wl1_ref,
                bb1_ref, gl2_ref, bl2_ref, Wl2_ref, bb2_ref, gl3_ref,
                bl3_ref, Wl3_ref, bb3_ref, Wx_ref, bx_ref, Wy_ref, by_ref,
                Wr_ref, br_ref, xx_ref, yy_ref, rot_ref):
    num = part_ref[0, 0:N, 0:HID] + part_ref[1, 0:N, 0:HID]
    s = part_ref[0, 0:N, HID:HID + 1] + part_ref[1, 0:N, HID:HID + 1]
    hf = jax.nn.relu(num / jnp.maximum(s, 1e-16) + b2_ref[...][None, :])
    gm = lax.broadcasted_iota(jnp.int32, (N, G), 1)
    M = jnp.where(gm == batch_ref[...][:, None], 1.0, 0.0)
    enc_sum = lax.dot_general(M, hf, (((0,), (0,)), ((), ())),
                              preferred_element_type=jnp.float32)
    cnt = jnp.sum(M, axis=0)
    enc = enc_sum / jnp.maximum(cnt, 1.0)[:, None]
    t = _bn_rows(enc, gl1_ref[...], bl1_ref[...])
    t = jax.nn.relu(jnp.dot(t, Wl1_ref[...]) + bb1_ref[...][None, :])
    t = _bn_rows(jnp.concatenate([t, enc], axis=1), gl2_ref[...], bl2_ref[...])
    t = jax.nn.relu(jnp.dot(t, Wl2_ref[...]) + bb2_ref[...][None, :])
    t = _bn_rows(jnp.concatenate([t, enc], axis=1), gl3_ref[...], bl3_ref[...])
    t = jax.nn.relu(jnp.dot(t, Wl3_ref[...]) + bb3_ref[...][None, :])
    xx_ref[...] = _softmax_rows(jnp.dot(t, Wx_ref[...]) + bx_ref[...][None, :])
    yy_ref[...] = _softmax_rows(jnp.dot(t, Wy_ref[...]) + by_ref[...][None, :])
    rot_ref[...] = _softmax_rows(jnp.dot(t, Wr_ref[...]) + br_ref[...][None, :])


_final_call = pl.pallas_call(
    _final_body,
    out_shape=[
        jax.ShapeDtypeStruct((G, ODX), jnp.float32),
        jax.ShapeDtypeStruct((G, ODY), jnp.float32),
        jax.ShapeDtypeStruct((G, 4), jnp.float32),
    ],
)


def kernel(x, edge_index, edge_attr, batch, W1, as1, ad1, We1, ae1, b1, g1,
           bt1, W2, as2, ad2, We2, ae2, b2, gl1, bl1, Wl1, bb1, gl2, bl2,
           Wl2, bb2, gl3, bl3, Wl3, bb3, Wx, bx, Wy, by, Wr, br):
    pad = E_PAD - E
    src = jnp.concatenate([edge_index[0], jnp.zeros((pad,), jnp.int32)])
    dst = jnp.concatenate([edge_index[1], jnp.zeros((pad,), jnp.int32)])
    srcp = src.reshape(ROWS, 128)
    dstp = dst.reshape(ROWS, 128)
    eaT = jnp.concatenate(
        [edge_attr, jnp.zeros((pad, 2), jnp.float32)], axis=0
    ).T.reshape(2, ROWS, 128)
    zz = jnp.zeros((N_ACC, ACC_W), jnp.float32)

    h1, an1, adn1, ea1p, ea2p = _prep_call(x, W1, as1, ad1, We1, ae1, We2,
                                           ae2, eaT)
    part1 = _edge_kernel(h1, an1, adn1, srcp, dstp, ea1p, zz)
    h2, an2, adn2 = _mid_call(part1, b1, g1, bt1, W2, as2, ad2)
    part2 = _edge_kernel(h2, an2, adn2, srcp, dstp, ea2p, zz)
    xx, yy, rot = _final_call(part2, b2, batch, gl1, bl1, Wl1, bb1, gl2,
                              bl2, Wl2, bb2, gl3, bl3, Wl3, bb3, Wx, bx,
                              Wy, by, Wr, br)
    return (xx, yy, rot)
